# baseline (speedup 1.0000x reference)
import functools

import jax
import jax.numpy as jnp
import numpy as np
from jax import lax
from jax.experimental import pallas as pl
from jax.experimental.pallas import tpu as pltpu
from jax.experimental.pallas import tpu_sc as plsc

NNODE = 10000
NPAD = 10240
NEDGE = 320000
EPAD = 335872
CW = 32
EROWS = EPAD // CW
BLK = 8
F = 128
NHEAD = 4
NC, NS, LANES = 2, 16, 16
ROWS_PER_TILE = NPAD // NS
P1_ROWS = EROWS // NS
P2_ROWS = EROWS // (NC * NS)


def _stage_a_body(x_ref, w_ref, aw_s_ref, aw_d_ref, h_ref, as_ref, ad_ref, c_ref):
    hb = jnp.dot(x_ref[...], w_ref[0], preferred_element_type=jnp.float32)
    h_ref[0] = hb
    asb = jnp.sum(hb * aw_s_ref[0, 0], axis=1)
    adb = jnp.sum(hb * aw_d_ref[0, 0], axis=1)
    as_ref[0, 0] = asb
    ad_ref[0, 0] = adb

    @pl.when(pl.program_id(1) == 0)
    def _():
        c_ref[...] = jnp.full((1, 2, 16), -1e30, jnp.float32)

    c_ref[0, 0, :] = jnp.maximum(c_ref[0, 0, :], jnp.max(asb))
    c_ref[0, 1, :] = jnp.maximum(c_ref[0, 1, :], jnp.max(adb))


_stage_a = pl.pallas_call(
    _stage_a_body,
    grid=(NHEAD, NPAD // 1024),
    in_specs=[
        pl.BlockSpec((1024, F), lambda h, i: (i, 0)),
        pl.BlockSpec((1, F, F), lambda h, i: (h, 0, 0)),
        pl.BlockSpec((1, 1, F), lambda h, i: (h, 0, 0)),
        pl.BlockSpec((1, 1, F), lambda h, i: (h, 0, 0)),
    ],
    out_specs=[
        pl.BlockSpec((1, 1024, F), lambda h, i: (h, i, 0)),
        pl.BlockSpec((1, 1, 1024), lambda h, i: (h, 0, i)),
        pl.BlockSpec((1, 1, 1024), lambda h, i: (h, 0, i)),
        pl.BlockSpec((1, 2, 16), lambda h, i: (h, 0, 0)),
    ],
    out_shape=[
        jax.ShapeDtypeStruct((NHEAD, NPAD, F), jnp.float32),
        jax.ShapeDtypeStruct((NHEAD, 1, NPAD), jnp.float32),
        jax.ShapeDtypeStruct((NHEAD, 1, NPAD), jnp.float32),
        jax.ShapeDtypeStruct((NHEAD, 2, 16), jnp.float32),
    ],
)


def _stage_b_body(p_ref, bs_ref, wo_ref, so_ref, do_ref,
                  h2_ref, as2_ref, ad2_ref, c_ref):
    acc = jnp.zeros((1024, F), jnp.float32)
    for h in range(NHEAD):
        ph = p_ref[h, 0] + p_ref[h, 1] + bs_ref[h][None, :]
        acc = acc + jnp.dot(ph, wo_ref[h], preferred_element_type=jnp.float32)
    h2_ref[...] = acc
    as2b = jnp.sum(acc * so_ref[0], axis=1)
    ad2b = jnp.sum(acc * do_ref[0], axis=1)
    as2_ref[0] = as2b
    ad2_ref[0] = ad2b

    @pl.when(pl.program_id(0) == 0)
    def _():
        c_ref[...] = jnp.full((2, 16), -1e30, jnp.float32)

    c_ref[0, :] = jnp.maximum(c_ref[0, :], jnp.max(as2b))
    c_ref[1, :] = jnp.maximum(c_ref[1, :], jnp.max(ad2b))


_stage_b = pl.pallas_call(
    _stage_b_body,
    grid=(NPAD // 1024,),
    in_specs=[pl.BlockSpec((NHEAD, NC, 1024, F), lambda i: (0, 0, i, 0))] + [
        pl.BlockSpec((NHEAD, F), lambda i: (0, 0)),
        pl.BlockSpec((NHEAD, F, F), lambda i: (0, 0, 0)),
        pl.BlockSpec((1, F), lambda i: (0, 0)),
        pl.BlockSpec((1, F), lambda i: (0, 0)),
    ],
    out_specs=[
        pl.BlockSpec((1024, F), lambda i: (i, 0)),
        pl.BlockSpec((1, 1024), lambda i: (0, i)),
        pl.BlockSpec((1, 1024), lambda i: (0, i)),
        pl.BlockSpec((2, 16), lambda i: (0, 0)),
    ],
    out_shape=[
        jax.ShapeDtypeStruct((NPAD, F), jnp.float32),
        jax.ShapeDtypeStruct((1, NPAD), jnp.float32),
        jax.ShapeDtypeStruct((1, NPAD), jnp.float32),
        jax.ShapeDtypeStruct((2, 16), jnp.float32),
    ],
)


def _stage_c_body(p_ref, bo_ref, out_ref):
    out_ref[...] = p_ref[0] + p_ref[1] + bo_ref[0][None, :]


_stage_c = pl.pallas_call(
    _stage_c_body,
    grid=(NNODE // 1000,),
    in_specs=[
        pl.BlockSpec((NC, 1000, F), lambda i: (0, i, 0)),
        pl.BlockSpec((1, F), lambda i: (0, 0)),
    ],
    out_specs=pl.BlockSpec((1000, F), lambda i: (i, 0)),
    out_shape=jax.ShapeDtypeStruct((NNODE, F), jnp.float32),
)


def _edge_body(*refs, nh):
    h_refs = refs[:nh]
    (asv_hbm, adv_hbm, cvec_hbm, src_hbm, dst_hbm, out_hbm, p_hbm,
     nb1, nb2, sstage, dstage, pstage, gb_a, gb_b, sc_a, sc_b, wbuf, cbuf,
     out_sh, den_sh, sem_a, sem_b, sem_sca, sem_scb, sem_s, sem_pw,
     sem_st) = refs[nh:]
    cid = lax.axis_index("c")
    sid = lax.axis_index("s")
    for hd in range(nh):
        _edge_one_head(h_refs[hd], asv_hbm.at[hd], adv_hbm.at[hd],
                       cvec_hbm.at[hd], src_hbm, dst_hbm,
                       out_hbm.at[hd], p_hbm.at[hd],
                       nb1, nb2, sstage, dstage, pstage, gb_a, gb_b, sc_a,
                       sc_b, wbuf, cbuf, out_sh, den_sh, sem_a, sem_b,
                       sem_sca, sem_scb, sem_s, sem_pw, sem_st, cid, sid)


def _edge_one_head(h_hbm, asv_hbm, adv_hbm, cvec_hbm, src_hbm, dst_hbm,
                   out_hbm, p_hbm,
                   nb1, nb2, sstage, dstage, pstage, gb_a, gb_b, sc_a, sc_b,
                   wbuf, cbuf,
                   out_sh, den_sh, sem_a, sem_b, sem_sca, sem_scb, sem_s,
                   sem_pw, sem_st, cid, sid):
    nvec = CW // LANES

    pltpu.sync_copy(asv_hbm, nb1)
    pltpu.sync_copy(adv_hbm, nb2)
    pltpu.sync_copy(cvec_hbm, cbuf)
    cshift = cbuf[pl.ds(0, LANES)] + cbuf[pl.ds(LANES, LANES)]

    zv = jnp.zeros((LANES,), jnp.float32)

    def _zrow(k, _):
        for l in range(8):
            sc_a[k, pl.ds(l * LANES, LANES)] = zv
        return 0

    lax.fori_loop(0, CW, _zrow, 0)
    for l in range(CW // LANES):
        wbuf[pl.ds(l * LANES, LANES)] = zv
    for k in range(ROWS_PER_TILE // CW):
        off = sid * ROWS_PER_TILE + k * CW
        pltpu.sync_copy(sc_a, out_sh.at[pl.ds(off, CW)])
    for k in range(ROWS_PER_TILE // CW):
        off = sid * ROWS_PER_TILE + k * CW
        pltpu.sync_copy(wbuf, den_sh.at[pl.ds(off, CW)])
    plsc.subcore_barrier()

    p1_base = sid * P1_ROWS
    p1_n = P1_ROWS // BLK
    pltpu.async_copy(src_hbm.at[pl.ds(p1_base, BLK)],
                     sstage.at[pl.ds(0, BLK)], sem_st)
    pltpu.async_copy(dst_hbm.at[pl.ds(p1_base, BLK)],
                     dstage.at[pl.ds(0, BLK)], sem_st)

    def _p1(blk, _):
        row8 = pl.multiple_of(p1_base + blk * BLK, 8)
        par = (blk % 2) * BLK
        pltpu.make_async_copy(src_hbm.at[pl.ds(row8, BLK)],
                              sstage.at[pl.ds(par, BLK)], sem_st).wait()
        pltpu.make_async_copy(dst_hbm.at[pl.ds(row8, BLK)],
                              dstage.at[pl.ds(par, BLK)], sem_st).wait()

        @pl.when(blk + 1 < p1_n)
        def _prefetch():
            row8n = pl.multiple_of(p1_base + (blk + 1) * BLK, 8)
            parn = ((blk + 1) % 2) * BLK
            pltpu.async_copy(src_hbm.at[pl.ds(row8n, BLK)],
                             sstage.at[pl.ds(parn, BLK)], sem_st)
            pltpu.async_copy(dst_hbm.at[pl.ds(row8n, BLK)],
                             dstage.at[pl.ds(parn, BLK)], sem_st)

        cps = []
        for jj in range(BLK):
            for v in range(nvec):
                sv = sstage[par + jj, pl.ds(v * LANES, LANES)]
                dv = dstage[par + jj, pl.ds(v * LANES, LANES)]
                e = (plsc.load_gather(nb1, [sv]) +
                     plsc.load_gather(nb2, [dv]))
                e = jnp.where(e > 0, e, 0.2 * e)
                pstage[jj, pl.ds(v * LANES, LANES)] = jnp.exp(e - cshift)
            cps.append(pltpu.async_copy(pstage.at[jj],
                                        den_sh.at[dstage.at[par + jj]],
                                        sem_s, add=True))
        cps.append(pltpu.async_copy(pstage.at[pl.ds(0, BLK)],
                                    p_hbm.at[pl.ds(row8, BLK)], sem_pw))
        for cp in cps:
            cp.wait()
        return 0

    lax.fori_loop(0, p1_n, _p1, 0)
    plsc.subcore_barrier()
    pltpu.sync_copy(den_sh, nb1)

    gbufs = (gb_a, gb_b)
    rbufs = (sc_a, sc_b)
    sems = (sem_a, sem_b)
    scsems = (sem_sca, sem_scb)
    p2_base = cid * (EROWS // 2) + sid * P2_ROWS
    p2_n = P2_ROWS // BLK
    for hbm, st in ((src_hbm, sstage), (dst_hbm, dstage), (p_hbm, pstage)):
        pltpu.async_copy(hbm.at[pl.ds(p2_base, BLK)],
                         st.at[pl.ds(0, BLK)], sem_st)

    def _p2(blk, _):
        row8 = pl.multiple_of(p2_base + blk * BLK, 8)
        par = (blk % 2) * BLK
        for hbm, st in ((src_hbm, sstage), (dst_hbm, dstage), (p_hbm, pstage)):
            pltpu.make_async_copy(hbm.at[pl.ds(row8, BLK)],
                                  st.at[pl.ds(par, BLK)], sem_st).wait()

        @pl.when(blk + 1 < p2_n)
        def _prefetch():
            row8n = pl.multiple_of(p2_base + (blk + 1) * BLK, 8)
            parn = ((blk + 1) % 2) * BLK
            for hbm, st in ((src_hbm, sstage), (dst_hbm, dstage),
                            (p_hbm, pstage)):
                pltpu.async_copy(hbm.at[pl.ds(row8n, BLK)],
                                 st.at[pl.ds(parn, BLK)], sem_st)

        sc_pending = [None, None]
        cp = pltpu.async_copy(h_hbm.at[sstage.at[par]], gbufs[0], sems[0])
        for jj in range(BLK):
            b = jj % 2
            if jj + 1 < BLK:
                nb_ = (jj + 1) % 2
                cp_next = pltpu.async_copy(h_hbm.at[sstage.at[par + jj + 1]],
                                           gbufs[nb_], sems[nb_])
            gbuf = gbufs[b]
            buf = rbufs[b]
            for v in range(nvec):
                dv = dstage[par + jj, pl.ds(v * LANES, LANES)]
                dn = plsc.load_gather(nb1, [dv])
                pv = pstage[par + jj, pl.ds(v * LANES, LANES)]
                wbuf[pl.ds(v * LANES, LANES)] = pv / (dn + 1e-16)
            cp.wait()
            if sc_pending[b] is not None:
                sc_pending[b].wait()
                sc_pending[b] = None

            def _scale(k, _c):
                wk = plsc.load_gather(wbuf, [jnp.full((LANES,), 0, jnp.int32) + k])
                for l in range(8):
                    buf[k, pl.ds(l * LANES, LANES)] = (
                        gbuf[k, pl.ds(l * LANES, LANES)] * wk)
                return 0

            lax.fori_loop(0, CW, _scale, 0)
            sc_pending[b] = pltpu.async_copy(buf, out_sh.at[dstage.at[par + jj]],
                                             scsems[b], add=True)
            if jj + 1 < BLK:
                cp = cp_next
        for d in sc_pending:
            if d is not None:
                d.wait()
        return 0

    lax.fori_loop(0, P2_ROWS // BLK, _p2, 0)
    plsc.subcore_barrier()

    for k in range(ROWS_PER_TILE // 128):
        off = sid * ROWS_PER_TILE + k * 128
        pltpu.sync_copy(out_sh.at[pl.ds(off, 128)],
                        out_hbm.at[cid, pl.ds(off, 128)])


@functools.cache
def _make_edge_pass(nh):
    return functools.partial(
        pl.kernel,
        mesh=plsc.VectorSubcoreMesh(core_axis_name="c", subcore_axis_name="s"),
        out_type=[
            jax.ShapeDtypeStruct((nh, NC, NPAD, F), jnp.float32),
            jax.ShapeDtypeStruct((nh, EROWS, CW), jnp.float32),
        ],
        compiler_params=pltpu.CompilerParams(needs_layout_passes=False),
        scratch_types=[
            pltpu.VMEM((NPAD,), jnp.float32),
            pltpu.VMEM((NPAD,), jnp.float32),
            pltpu.VMEM((2 * BLK, CW), jnp.int32),
            pltpu.VMEM((2 * BLK, CW), jnp.int32),
            pltpu.VMEM((2 * BLK, CW), jnp.float32),
            pltpu.VMEM((CW, F), jnp.float32),
            pltpu.VMEM((CW, F), jnp.float32),
            pltpu.VMEM((CW, F), jnp.float32),
            pltpu.VMEM((CW, F), jnp.float32),
            pltpu.VMEM((CW,), jnp.float32),
            pltpu.VMEM((2 * LANES,), jnp.float32),
            pltpu.VMEM_SHARED((NPAD, F), jnp.float32),
            pltpu.VMEM_SHARED((NPAD,), jnp.float32),
            pltpu.SemaphoreType.DMA,
            pltpu.SemaphoreType.DMA,
            pltpu.SemaphoreType.DMA,
            pltpu.SemaphoreType.DMA,
            pltpu.SemaphoreType.DMA,
            pltpu.SemaphoreType.DMA,
            pltpu.SemaphoreType.DMA,
        ],
    )(functools.partial(_edge_body, nh=nh))


def kernel(atom_features, edge_index, Ws, att_s, att_d, bs, Wo, att_so, att_do, bo):
    x_pad = jnp.pad(atom_features, ((0, NPAD - NNODE), (0, 0)))
    loop_idx = jnp.arange(NNODE, dtype=jnp.int32)
    npad_e = EPAD - NEDGE - NNODE
    pad_src = (jnp.arange(npad_e, dtype=jnp.int32) * 37) % NNODE
    pad_dst = NNODE + (jnp.arange(npad_e, dtype=jnp.int32) % (NPAD - NNODE))
    src = jnp.concatenate([edge_index[0], loop_idx, pad_src]).reshape(EROWS, CW)
    dst = jnp.concatenate([edge_index[1], loop_idx, pad_dst]).reshape(EROWS, CW)

    h_all, as_all, ad_all, c_all = _stage_a(x_pad, Ws, att_s.reshape(NHEAD, 1, F),
                                            att_d.reshape(NHEAD, 1, F))
    parts = _make_edge_pass(NHEAD)(
        h_all[0], h_all[1], h_all[2], h_all[3],
        as_all[:, 0, :], ad_all[:, 0, :], c_all.reshape(NHEAD, 2 * LANES),
        src, dst)[0]
    wo4 = Wo.reshape(NHEAD, F, F)
    h2, as2, ad2, c2 = _stage_b(parts, bs, wo4, att_so.reshape(1, F),
                                att_do.reshape(1, F))
    p2 = _make_edge_pass(1)(h2, as2, ad2, c2.reshape(1, 2 * LANES),
                            src, dst)[0]
    return _stage_c(p2.reshape(NC, NPAD, F), bo.reshape(1, F))


# speedup vs baseline: 1.9657x; 1.9657x over previous
import functools

import jax
import jax.numpy as jnp
import numpy as np
from jax import lax
from jax.experimental import pallas as pl
from jax.experimental.pallas import tpu as pltpu
from jax.experimental.pallas import tpu_sc as plsc

NNODE = 10000
NPAD = 10240
NEDGE = 320000
EPAD = 344064
CW = 64
EROWS = EPAD // CW
BLK = 8
F = 128
NHEAD = 4
NC, NS, LANES = 2, 16, 16
ROWS_PER_TILE = NPAD // NS
P1_ROWS = EROWS // NS
P2_ROWS = EROWS // (NC * NS)


def _stage_a_body(x_ref, w_ref, aw_s_ref, aw_d_ref, h_ref, as_ref, ad_ref, c_ref):
    hb = jnp.dot(x_ref[...], w_ref[0], preferred_element_type=jnp.float32)
    h_ref[0] = hb
    asb = jnp.sum(hb * aw_s_ref[0, 0], axis=1)
    adb = jnp.sum(hb * aw_d_ref[0, 0], axis=1)
    as_ref[0, 0] = asb
    ad_ref[0, 0] = adb

    @pl.when(pl.program_id(1) == 0)
    def _():
        c_ref[...] = jnp.full((1, 2, 16), -1e30, jnp.float32)

    c_ref[0, 0, :] = jnp.maximum(c_ref[0, 0, :], jnp.max(asb))
    c_ref[0, 1, :] = jnp.maximum(c_ref[0, 1, :], jnp.max(adb))


_stage_a = pl.pallas_call(
    _stage_a_body,
    grid=(NHEAD, NPAD // 1024),
    in_specs=[
        pl.BlockSpec((1024, F), lambda h, i: (i, 0)),
        pl.BlockSpec((1, F, F), lambda h, i: (h, 0, 0)),
        pl.BlockSpec((1, 1, F), lambda h, i: (h, 0, 0)),
        pl.BlockSpec((1, 1, F), lambda h, i: (h, 0, 0)),
    ],
    out_specs=[
        pl.BlockSpec((1, 1024, F), lambda h, i: (h, i, 0)),
        pl.BlockSpec((1, 1, 1024), lambda h, i: (h, 0, i)),
        pl.BlockSpec((1, 1, 1024), lambda h, i: (h, 0, i)),
        pl.BlockSpec((1, 2, 16), lambda h, i: (h, 0, 0)),
    ],
    out_shape=[
        jax.ShapeDtypeStruct((NHEAD, NPAD, F), jnp.float32),
        jax.ShapeDtypeStruct((NHEAD, 1, NPAD), jnp.float32),
        jax.ShapeDtypeStruct((NHEAD, 1, NPAD), jnp.float32),
        jax.ShapeDtypeStruct((NHEAD, 2, 16), jnp.float32),
    ],
)


def _stage_b_body(p_ref, bs_ref, wo_ref, so_ref, do_ref,
                  h2_ref, as2_ref, ad2_ref, c_ref):
    acc = jnp.zeros((1024, F), jnp.float32)
    for h in range(NHEAD):
        ph = p_ref[h, 0] + p_ref[h, 1] + bs_ref[h][None, :]
        acc = acc + jnp.dot(ph, wo_ref[h], preferred_element_type=jnp.float32)
    h2_ref[...] = acc
    as2b = jnp.sum(acc * so_ref[0], axis=1)
    ad2b = jnp.sum(acc * do_ref[0], axis=1)
    as2_ref[0] = as2b
    ad2_ref[0] = ad2b

    @pl.when(pl.program_id(0) == 0)
    def _():
        c_ref[...] = jnp.full((2, 16), -1e30, jnp.float32)

    c_ref[0, :] = jnp.maximum(c_ref[0, :], jnp.max(as2b))
    c_ref[1, :] = jnp.maximum(c_ref[1, :], jnp.max(ad2b))


_stage_b = pl.pallas_call(
    _stage_b_body,
    grid=(NPAD // 1024,),
    in_specs=[pl.BlockSpec((NHEAD, NC, 1024, F), lambda i: (0, 0, i, 0))] + [
        pl.BlockSpec((NHEAD, F), lambda i: (0, 0)),
        pl.BlockSpec((NHEAD, F, F), lambda i: (0, 0, 0)),
        pl.BlockSpec((1, F), lambda i: (0, 0)),
        pl.BlockSpec((1, F), lambda i: (0, 0)),
    ],
    out_specs=[
        pl.BlockSpec((1024, F), lambda i: (i, 0)),
        pl.BlockSpec((1, 1024), lambda i: (0, i)),
        pl.BlockSpec((1, 1024), lambda i: (0, i)),
        pl.BlockSpec((2, 16), lambda i: (0, 0)),
    ],
    out_shape=[
        jax.ShapeDtypeStruct((NPAD, F), jnp.float32),
        jax.ShapeDtypeStruct((1, NPAD), jnp.float32),
        jax.ShapeDtypeStruct((1, NPAD), jnp.float32),
        jax.ShapeDtypeStruct((2, 16), jnp.float32),
    ],
)


def _stage_c_body(p_ref, bo_ref, out_ref):
    out_ref[...] = p_ref[0] + p_ref[1] + bo_ref[0][None, :]


_stage_c = pl.pallas_call(
    _stage_c_body,
    grid=(NNODE // 1000,),
    in_specs=[
        pl.BlockSpec((NC, 1000, F), lambda i: (0, i, 0)),
        pl.BlockSpec((1, F), lambda i: (0, 0)),
    ],
    out_specs=pl.BlockSpec((1000, F), lambda i: (i, 0)),
    out_shape=jax.ShapeDtypeStruct((NNODE, F), jnp.float32),
)


def _edge_body(*refs, nh):
    h_refs = refs[:nh]
    (asv_hbm, adv_hbm, cvec_hbm, src_hbm, dst_hbm, out_hbm, p_hbm,
     nb1, nb2, sstage, dstage, pstage, sc_a, sc_b, wbuf, cbuf,
     out_sh, den_sh, sem_a, sem_b, sem_sca, sem_scb, sem_s, sem_pw,
     sem_st) = refs[nh:]
    cid = lax.axis_index("c")
    sid = lax.axis_index("s")
    for hd in range(nh):
        _edge_one_head(h_refs[hd], asv_hbm.at[hd], adv_hbm.at[hd],
                       cvec_hbm.at[hd], src_hbm, dst_hbm,
                       out_hbm.at[hd], p_hbm.at[hd],
                       nb1, nb2, sstage, dstage, pstage, sc_a,
                       sc_b, wbuf, cbuf, out_sh, den_sh, sem_a, sem_b,
                       sem_sca, sem_scb, sem_s, sem_pw, sem_st, cid, sid)


def _edge_one_head(h_hbm, asv_hbm, adv_hbm, cvec_hbm, src_hbm, dst_hbm,
                   out_hbm, p_hbm,
                   nb1, nb2, sstage, dstage, pstage, sc_a, sc_b,
                   wbuf, cbuf,
                   out_sh, den_sh, sem_a, sem_b, sem_sca, sem_scb, sem_s,
                   sem_pw, sem_st, cid, sid):
    nvec = CW // LANES

    pltpu.sync_copy(asv_hbm, nb1)
    pltpu.sync_copy(adv_hbm, nb2)
    pltpu.sync_copy(cvec_hbm, cbuf)
    cshift = cbuf[pl.ds(0, LANES)] + cbuf[pl.ds(LANES, LANES)]

    zv = jnp.zeros((LANES,), jnp.float32)

    def _zrow(k, _):
        for l in range(8):
            sc_a[k, pl.ds(l * LANES, LANES)] = zv
        return 0

    lax.fori_loop(0, CW, _zrow, 0)
    for l in range(CW // LANES):
        wbuf[pl.ds(l * LANES, LANES)] = zv
    for k in range(ROWS_PER_TILE // CW):
        off = sid * ROWS_PER_TILE + k * CW
        pltpu.sync_copy(sc_a, out_sh.at[pl.ds(off, CW)])
    for k in range(ROWS_PER_TILE // CW):
        off = sid * ROWS_PER_TILE + k * CW
        pltpu.sync_copy(wbuf, den_sh.at[pl.ds(off, CW)])
    plsc.subcore_barrier()

    p1_base = sid * P1_ROWS
    p1_n = P1_ROWS // BLK
    pltpu.async_copy(src_hbm.at[pl.ds(p1_base, BLK)],
                     sstage.at[pl.ds(0, BLK)], sem_st)
    pltpu.async_copy(dst_hbm.at[pl.ds(p1_base, BLK)],
                     dstage.at[pl.ds(0, BLK)], sem_st)

    def _p1(blk, _):
        row8 = pl.multiple_of(p1_base + blk * BLK, 8)
        par = (blk % 2) * BLK
        pltpu.make_async_copy(src_hbm.at[pl.ds(row8, BLK)],
                              sstage.at[pl.ds(par, BLK)], sem_st).wait()
        pltpu.make_async_copy(dst_hbm.at[pl.ds(row8, BLK)],
                              dstage.at[pl.ds(par, BLK)], sem_st).wait()

        @pl.when(blk + 1 < p1_n)
        def _prefetch():
            row8n = pl.multiple_of(p1_base + (blk + 1) * BLK, 8)
            parn = ((blk + 1) % 2) * BLK
            pltpu.async_copy(src_hbm.at[pl.ds(row8n, BLK)],
                             sstage.at[pl.ds(parn, BLK)], sem_st)
            pltpu.async_copy(dst_hbm.at[pl.ds(row8n, BLK)],
                             dstage.at[pl.ds(parn, BLK)], sem_st)

        cps = []
        for jj in range(BLK):
            for v in range(nvec):
                sv = sstage[par + jj, pl.ds(v * LANES, LANES)]
                dv = dstage[par + jj, pl.ds(v * LANES, LANES)]
                e = (plsc.load_gather(nb1, [sv]) +
                     plsc.load_gather(nb2, [dv]))
                e = jnp.where(e > 0, e, 0.2 * e)
                pstage[jj, pl.ds(v * LANES, LANES)] = jnp.exp(e - cshift)
            cps.append(pltpu.async_copy(pstage.at[jj],
                                        den_sh.at[dstage.at[par + jj]],
                                        sem_s, add=True))
        cps.append(pltpu.async_copy(pstage.at[pl.ds(0, BLK)],
                                    p_hbm.at[pl.ds(row8, BLK)], sem_pw))
        for cp in cps:
            cp.wait()
        return 0

    lax.fori_loop(0, p1_n, _p1, 0)
    plsc.subcore_barrier()
    pltpu.sync_copy(den_sh, nb1)

    gbufs = (sc_a, sc_b)
    rbufs = (sc_a, sc_b)
    sems = (sem_a, sem_b)
    scsems = (sem_sca, sem_scb)
    p2_base = cid * (EROWS // 2) + sid * P2_ROWS
    p2_n = P2_ROWS // BLK
    for hbm, st in ((src_hbm, sstage), (dst_hbm, dstage), (p_hbm, pstage)):
        pltpu.async_copy(hbm.at[pl.ds(p2_base, BLK)],
                         st.at[pl.ds(0, BLK)], sem_st)

    def _p2(blk, _):
        row8 = pl.multiple_of(p2_base + blk * BLK, 8)
        par = (blk % 2) * BLK
        for hbm, st in ((src_hbm, sstage), (dst_hbm, dstage), (p_hbm, pstage)):
            pltpu.make_async_copy(hbm.at[pl.ds(row8, BLK)],
                                  st.at[pl.ds(par, BLK)], sem_st).wait()

        @pl.when(blk + 1 < p2_n)
        def _prefetch():
            row8n = pl.multiple_of(p2_base + (blk + 1) * BLK, 8)
            parn = ((blk + 1) % 2) * BLK
            for hbm, st in ((src_hbm, sstage), (dst_hbm, dstage),
                            (p_hbm, pstage)):
                pltpu.async_copy(hbm.at[pl.ds(row8n, BLK)],
                                 st.at[pl.ds(parn, BLK)], sem_st)

        sc_pending = [None, None]
        cp = pltpu.async_copy(h_hbm.at[sstage.at[par]], gbufs[0], sems[0])
        for jj in range(BLK):
            b = jj % 2
            if jj + 1 < BLK:
                nb_ = (jj + 1) % 2
                if sc_pending[nb_] is not None:
                    sc_pending[nb_].wait()
                    sc_pending[nb_] = None
                cp_next = pltpu.async_copy(h_hbm.at[sstage.at[par + jj + 1]],
                                           gbufs[nb_], sems[nb_])
            buf = rbufs[b]
            for v in range(nvec):
                dv = dstage[par + jj, pl.ds(v * LANES, LANES)]
                dn = plsc.load_gather(nb1, [dv])
                pv = pstage[par + jj, pl.ds(v * LANES, LANES)]
                wbuf[pl.ds(v * LANES, LANES)] = pv / (dn + 1e-16)
            cp.wait()

            def _scale(k, _c):
                wk = plsc.load_gather(wbuf, [jnp.full((LANES,), 0, jnp.int32) + k])
                for l in range(8):
                    buf[k, pl.ds(l * LANES, LANES)] = (
                        buf[k, pl.ds(l * LANES, LANES)] * wk)
                return 0

            lax.fori_loop(0, CW, _scale, 0)
            sc_pending[b] = pltpu.async_copy(buf, out_sh.at[dstage.at[par + jj]],
                                             scsems[b], add=True)
            if jj + 1 < BLK:
                cp = cp_next
        for d in sc_pending:
            if d is not None:
                d.wait()
        return 0

    lax.fori_loop(0, P2_ROWS // BLK, _p2, 0)
    plsc.subcore_barrier()

    for k in range(ROWS_PER_TILE // 128):
        off = sid * ROWS_PER_TILE + k * 128
        pltpu.sync_copy(out_sh.at[pl.ds(off, 128)],
                        out_hbm.at[cid, pl.ds(off, 128)])


@functools.cache
def _make_edge_pass(nh):
    return functools.partial(
        pl.kernel,
        mesh=plsc.VectorSubcoreMesh(core_axis_name="c", subcore_axis_name="s"),
        out_type=[
            jax.ShapeDtypeStruct((nh, NC, NPAD, F), jnp.float32),
            jax.ShapeDtypeStruct((nh, EROWS, CW), jnp.float32),
        ],
        compiler_params=pltpu.CompilerParams(needs_layout_passes=False),
        scratch_types=[
            pltpu.VMEM((NPAD,), jnp.float32),
            pltpu.VMEM((NPAD,), jnp.float32),
            pltpu.VMEM((2 * BLK, CW), jnp.int32),
            pltpu.VMEM((2 * BLK, CW), jnp.int32),
            pltpu.VMEM((2 * BLK, CW), jnp.float32),
            pltpu.VMEM((CW, F), jnp.float32),
            pltpu.VMEM((CW, F), jnp.float32),
            pltpu.VMEM((CW,), jnp.float32),
            pltpu.VMEM((2 * LANES,), jnp.float32),
            pltpu.VMEM_SHARED((NPAD, F), jnp.float32),
            pltpu.VMEM_SHARED((NPAD,), jnp.float32),
            pltpu.SemaphoreType.DMA,
            pltpu.SemaphoreType.DMA,
            pltpu.SemaphoreType.DMA,
            pltpu.SemaphoreType.DMA,
            pltpu.SemaphoreType.DMA,
            pltpu.SemaphoreType.DMA,
            pltpu.SemaphoreType.DMA,
        ],
    )(functools.partial(_edge_body, nh=nh))


def kernel(atom_features, edge_index, Ws, att_s, att_d, bs, Wo, att_so, att_do, bo):
    x_pad = jnp.pad(atom_features, ((0, NPAD - NNODE), (0, 0)))
    loop_idx = jnp.arange(NNODE, dtype=jnp.int32)
    npad_e = EPAD - NEDGE - NNODE
    pad_src = (jnp.arange(npad_e, dtype=jnp.int32) * 37) % NNODE
    pad_dst = NNODE + (jnp.arange(npad_e, dtype=jnp.int32) % (NPAD - NNODE))
    src = jnp.concatenate([edge_index[0], loop_idx, pad_src]).reshape(EROWS, CW)
    dst = jnp.concatenate([edge_index[1], loop_idx, pad_dst]).reshape(EROWS, CW)

    h_all, as_all, ad_all, c_all = _stage_a(x_pad, Ws, att_s.reshape(NHEAD, 1, F),
                                            att_d.reshape(NHEAD, 1, F))
    parts = _make_edge_pass(NHEAD)(
        h_all[0], h_all[1], h_all[2], h_all[3],
        as_all[:, 0, :], ad_all[:, 0, :], c_all.reshape(NHEAD, 2 * LANES),
        src, dst)[0]
    wo4 = Wo.reshape(NHEAD, F, F)
    h2, as2, ad2, c2 = _stage_b(parts, bs, wo4, att_so.reshape(1, F),
                                att_do.reshape(1, F))
    p2 = _make_edge_pass(1)(h2, as2, ad2, c2.reshape(1, 2 * LANES),
                            src, dst)[0]
    return _stage_c(p2.reshape(NC, NPAD, F), bo.reshape(1, F))


# speedup vs baseline: 2.0168x; 1.0260x over previous
import functools

import jax
import jax.numpy as jnp
import numpy as np
from jax import lax
from jax.experimental import pallas as pl
from jax.experimental.pallas import tpu as pltpu
from jax.experimental.pallas import tpu_sc as plsc

NNODE = 10000
NPAD = 10240
NEDGE = 320000
EPAD = 344064
CW = 64
EROWS = EPAD // CW
BLK = 8
F = 128
NHEAD = 4
NC, NS, LANES = 2, 16, 16
ROWS_PER_TILE = NPAD // NS
P1_ROWS = EROWS // NS
P2_ROWS = EROWS // (NC * NS)


def _stage_a_body(x_ref, w_ref, aw_s_ref, aw_d_ref, h_ref, as_ref, ad_ref, c_ref):
    hb = jnp.dot(x_ref[...], w_ref[0], preferred_element_type=jnp.float32)
    h_ref[0] = hb
    asb = jnp.sum(hb * aw_s_ref[0, 0], axis=1)
    adb = jnp.sum(hb * aw_d_ref[0, 0], axis=1)
    as_ref[0, 0] = asb
    ad_ref[0, 0] = adb

    @pl.when(pl.program_id(1) == 0)
    def _():
        c_ref[...] = jnp.full((1, 2, 16), -1e30, jnp.float32)

    c_ref[0, 0, :] = jnp.maximum(c_ref[0, 0, :], jnp.max(asb))
    c_ref[0, 1, :] = jnp.maximum(c_ref[0, 1, :], jnp.max(adb))


_stage_a = pl.pallas_call(
    _stage_a_body,
    grid=(NHEAD, NPAD // 1024),
    in_specs=[
        pl.BlockSpec((1024, F), lambda h, i: (i, 0)),
        pl.BlockSpec((1, F, F), lambda h, i: (h, 0, 0)),
        pl.BlockSpec((1, 1, F), lambda h, i: (h, 0, 0)),
        pl.BlockSpec((1, 1, F), lambda h, i: (h, 0, 0)),
    ],
    out_specs=[
        pl.BlockSpec((1, 1024, F), lambda h, i: (h, i, 0)),
        pl.BlockSpec((1, 1, 1024), lambda h, i: (h, 0, i)),
        pl.BlockSpec((1, 1, 1024), lambda h, i: (h, 0, i)),
        pl.BlockSpec((1, 2, 16), lambda h, i: (h, 0, 0)),
    ],
    out_shape=[
        jax.ShapeDtypeStruct((NHEAD, NPAD, F), jnp.float32),
        jax.ShapeDtypeStruct((NHEAD, 1, NPAD), jnp.float32),
        jax.ShapeDtypeStruct((NHEAD, 1, NPAD), jnp.float32),
        jax.ShapeDtypeStruct((NHEAD, 2, 16), jnp.float32),
    ],
)


def _stage_b_body(p_ref, bs_ref, wo_ref, so_ref, do_ref,
                  h2_ref, as2_ref, ad2_ref, c_ref):
    acc = jnp.zeros((1024, F), jnp.float32)
    for h in range(NHEAD):
        ph = p_ref[h, 0] + p_ref[h, 1] + bs_ref[h][None, :]
        acc = acc + jnp.dot(ph, wo_ref[h], preferred_element_type=jnp.float32)
    h2_ref[...] = acc
    as2b = jnp.sum(acc * so_ref[0], axis=1)
    ad2b = jnp.sum(acc * do_ref[0], axis=1)
    as2_ref[0] = as2b
    ad2_ref[0] = ad2b

    @pl.when(pl.program_id(0) == 0)
    def _():
        c_ref[...] = jnp.full((2, 16), -1e30, jnp.float32)

    c_ref[0, :] = jnp.maximum(c_ref[0, :], jnp.max(as2b))
    c_ref[1, :] = jnp.maximum(c_ref[1, :], jnp.max(ad2b))


_stage_b = pl.pallas_call(
    _stage_b_body,
    grid=(NPAD // 1024,),
    in_specs=[pl.BlockSpec((NHEAD, NC, 1024, F), lambda i: (0, 0, i, 0))] + [
        pl.BlockSpec((NHEAD, F), lambda i: (0, 0)),
        pl.BlockSpec((NHEAD, F, F), lambda i: (0, 0, 0)),
        pl.BlockSpec((1, F), lambda i: (0, 0)),
        pl.BlockSpec((1, F), lambda i: (0, 0)),
    ],
    out_specs=[
        pl.BlockSpec((1024, F), lambda i: (i, 0)),
        pl.BlockSpec((1, 1024), lambda i: (0, i)),
        pl.BlockSpec((1, 1024), lambda i: (0, i)),
        pl.BlockSpec((2, 16), lambda i: (0, 0)),
    ],
    out_shape=[
        jax.ShapeDtypeStruct((NPAD, F), jnp.float32),
        jax.ShapeDtypeStruct((1, NPAD), jnp.float32),
        jax.ShapeDtypeStruct((1, NPAD), jnp.float32),
        jax.ShapeDtypeStruct((2, 16), jnp.float32),
    ],
)


def _stage_c_body(p_ref, bo_ref, out_ref):
    out_ref[...] = p_ref[0] + p_ref[1] + bo_ref[0][None, :]


_stage_c = pl.pallas_call(
    _stage_c_body,
    grid=(NNODE // 1000,),
    in_specs=[
        pl.BlockSpec((NC, 1000, F), lambda i: (0, i, 0)),
        pl.BlockSpec((1, F), lambda i: (0, 0)),
    ],
    out_specs=pl.BlockSpec((1000, F), lambda i: (i, 0)),
    out_shape=jax.ShapeDtypeStruct((NNODE, F), jnp.float32),
)


def _edge_body(*refs, nh):
    h_refs = refs[:nh]
    (asv_hbm, adv_hbm, cvec_hbm, src_hbm, dst_hbm, out_hbm, p_hbm,
     nb1, nb2, sstage, dstage, pstage, sc_a, sc_b, wbuf, cbuf,
     out_sh, den_sh, sem_a, sem_b, sem_sca, sem_scb, sem_s, sem_pw,
     sem_st) = refs[nh:]
    cid = lax.axis_index("c")
    sid = lax.axis_index("s")
    for hd in range(nh):
        _edge_one_head(h_refs[hd], asv_hbm.at[hd], adv_hbm.at[hd],
                       cvec_hbm.at[hd], src_hbm, dst_hbm,
                       out_hbm.at[hd], p_hbm.at[hd],
                       nb1, nb2, sstage, dstage, pstage, sc_a,
                       sc_b, wbuf, cbuf, out_sh, den_sh, sem_a, sem_b,
                       sem_sca, sem_scb, sem_s, sem_pw, sem_st, cid, sid)


def _edge_one_head(h_hbm, asv_hbm, adv_hbm, cvec_hbm, src_hbm, dst_hbm,
                   out_hbm, p_hbm,
                   nb1, nb2, sstage, dstage, pstage, sc_a, sc_b,
                   wbuf, cbuf,
                   out_sh, den_sh, sem_a, sem_b, sem_sca, sem_scb, sem_s,
                   sem_pw, sem_st, cid, sid):
    nvec = CW // LANES

    pltpu.sync_copy(asv_hbm, nb1)
    pltpu.sync_copy(adv_hbm, nb2)
    pltpu.sync_copy(cvec_hbm, cbuf)
    cshift = cbuf[pl.ds(0, LANES)] + cbuf[pl.ds(LANES, LANES)]

    zv = jnp.zeros((LANES,), jnp.float32)

    def _zrow(k, _):
        for l in range(8):
            sc_a[k, pl.ds(l * LANES, LANES)] = zv
        return 0

    lax.fori_loop(0, CW, _zrow, 0)
    for l in range(CW // LANES):
        wbuf[pl.ds(l * LANES, LANES)] = zv
    for k in range(ROWS_PER_TILE // CW):
        off = sid * ROWS_PER_TILE + k * CW
        pltpu.sync_copy(sc_a, out_sh.at[pl.ds(off, CW)])
    for k in range(ROWS_PER_TILE // CW):
        off = sid * ROWS_PER_TILE + k * CW
        pltpu.sync_copy(wbuf, den_sh.at[pl.ds(off, CW)])
    plsc.subcore_barrier()

    p1_base = sid * P1_ROWS
    p1_n = P1_ROWS // BLK
    pltpu.async_copy(src_hbm.at[pl.ds(p1_base, BLK)],
                     sstage.at[pl.ds(0, BLK)], sem_st)
    pltpu.async_copy(dst_hbm.at[pl.ds(p1_base, BLK)],
                     dstage.at[pl.ds(0, BLK)], sem_st)

    def _p1(blk, _):
        row8 = pl.multiple_of(p1_base + blk * BLK, 8)
        par = (blk % 2) * BLK
        pltpu.make_async_copy(src_hbm.at[pl.ds(row8, BLK)],
                              sstage.at[pl.ds(par, BLK)], sem_st).wait()
        pltpu.make_async_copy(dst_hbm.at[pl.ds(row8, BLK)],
                              dstage.at[pl.ds(par, BLK)], sem_st).wait()

        @pl.when(blk + 1 < p1_n)
        def _prefetch():
            row8n = pl.multiple_of(p1_base + (blk + 1) * BLK, 8)
            parn = ((blk + 1) % 2) * BLK
            pltpu.async_copy(src_hbm.at[pl.ds(row8n, BLK)],
                             sstage.at[pl.ds(parn, BLK)], sem_st)
            pltpu.async_copy(dst_hbm.at[pl.ds(row8n, BLK)],
                             dstage.at[pl.ds(parn, BLK)], sem_st)

        cps = []
        for jj in range(BLK):
            for v in range(nvec):
                sv = sstage[par + jj, pl.ds(v * LANES, LANES)]
                dv = dstage[par + jj, pl.ds(v * LANES, LANES)]
                e = (plsc.load_gather(nb1, [sv]) +
                     plsc.load_gather(nb2, [dv]))
                e = jnp.where(e > 0, e, 0.2 * e)
                pstage[jj, pl.ds(v * LANES, LANES)] = jnp.exp(e - cshift)
            cps.append(pltpu.async_copy(pstage.at[jj],
                                        den_sh.at[dstage.at[par + jj]],
                                        sem_s, add=True))
        cps.append(pltpu.async_copy(pstage.at[pl.ds(0, BLK)],
                                    p_hbm.at[pl.ds(row8, BLK)], sem_pw))
        for cp in cps:
            cp.wait()
        return 0

    lax.fori_loop(0, p1_n, _p1, 0)
    plsc.subcore_barrier()
    pltpu.sync_copy(den_sh, nb1)

    gbufs = (sc_a, sc_b)
    rbufs = (sc_a, sc_b)
    sems = (sem_a, sem_b)
    scsems = (sem_sca, sem_scb)
    p2_base = cid * (EROWS // 2) + sid * P2_ROWS
    p2_n = P2_ROWS // BLK
    for hbm, st in ((src_hbm, sstage), (dst_hbm, dstage), (p_hbm, pstage)):
        pltpu.async_copy(hbm.at[pl.ds(p2_base, BLK)],
                         st.at[pl.ds(0, BLK)], sem_st)

    def _p2(blk, _):
        row8 = pl.multiple_of(p2_base + blk * BLK, 8)
        par = (blk % 2) * BLK
        for hbm, st in ((src_hbm, sstage), (dst_hbm, dstage), (p_hbm, pstage)):
            pltpu.make_async_copy(hbm.at[pl.ds(row8, BLK)],
                                  st.at[pl.ds(par, BLK)], sem_st).wait()

        @pl.when(blk + 1 < p2_n)
        def _prefetch():
            row8n = pl.multiple_of(p2_base + (blk + 1) * BLK, 8)
            parn = ((blk + 1) % 2) * BLK
            for hbm, st in ((src_hbm, sstage), (dst_hbm, dstage),
                            (p_hbm, pstage)):
                pltpu.async_copy(hbm.at[pl.ds(row8n, BLK)],
                                 st.at[pl.ds(parn, BLK)], sem_st)

        sc_pending = [None, None]
        cp = pltpu.async_copy(h_hbm.at[sstage.at[par]], gbufs[0], sems[0])
        for jj in range(BLK):
            b = jj % 2
            if jj + 1 < BLK:
                nb_ = (jj + 1) % 2
                if sc_pending[nb_] is not None:
                    sc_pending[nb_].wait()
                    sc_pending[nb_] = None
                cp_next = pltpu.async_copy(h_hbm.at[sstage.at[par + jj + 1]],
                                           gbufs[nb_], sems[nb_])
            buf = rbufs[b]
            for v in range(nvec):
                dv = dstage[par + jj, pl.ds(v * LANES, LANES)]
                dn = plsc.load_gather(nb1, [dv])
                pv = pstage[par + jj, pl.ds(v * LANES, LANES)]
                wbuf[pl.ds(v * LANES, LANES)] = pv / (dn + 1e-16)
            cp.wait()

            def _scale(k2, _c):
                for r in range(2):
                    k = k2 * 2 + r
                    wk = plsc.load_gather(
                        wbuf, [jnp.full((LANES,), r, jnp.int32) + k2 * 2])
                    for l in range(8):
                        buf[k, pl.ds(l * LANES, LANES)] = (
                            buf[k, pl.ds(l * LANES, LANES)] * wk)
                return 0

            lax.fori_loop(0, CW // 2, _scale, 0)
            sc_pending[b] = pltpu.async_copy(buf, out_sh.at[dstage.at[par + jj]],
                                             scsems[b], add=True)
            if jj + 1 < BLK:
                cp = cp_next
        for d in sc_pending:
            if d is not None:
                d.wait()
        return 0

    lax.fori_loop(0, P2_ROWS // BLK, _p2, 0)
    plsc.subcore_barrier()

    for k in range(ROWS_PER_TILE // 128):
        off = sid * ROWS_PER_TILE + k * 128
        pltpu.sync_copy(out_sh.at[pl.ds(off, 128)],
                        out_hbm.at[cid, pl.ds(off, 128)])


@functools.cache
def _make_edge_pass(nh):
    return functools.partial(
        pl.kernel,
        mesh=plsc.VectorSubcoreMesh(core_axis_name="c", subcore_axis_name="s"),
        out_type=[
            jax.ShapeDtypeStruct((nh, NC, NPAD, F), jnp.float32),
            jax.ShapeDtypeStruct((nh, EROWS, CW), jnp.float32),
        ],
        compiler_params=pltpu.CompilerParams(needs_layout_passes=False),
        scratch_types=[
            pltpu.VMEM((NPAD,), jnp.float32),
            pltpu.VMEM((NPAD,), jnp.float32),
            pltpu.VMEM((2 * BLK, CW), jnp.int32),
            pltpu.VMEM((2 * BLK, CW), jnp.int32),
            pltpu.VMEM((2 * BLK, CW), jnp.float32),
            pltpu.VMEM((CW, F), jnp.float32),
            pltpu.VMEM((CW, F), jnp.float32),
            pltpu.VMEM((CW,), jnp.float32),
            pltpu.VMEM((2 * LANES,), jnp.float32),
            pltpu.VMEM_SHARED((NPAD, F), jnp.float32),
            pltpu.VMEM_SHARED((NPAD,), jnp.float32),
            pltpu.SemaphoreType.DMA,
            pltpu.SemaphoreType.DMA,
            pltpu.SemaphoreType.DMA,
            pltpu.SemaphoreType.DMA,
            pltpu.SemaphoreType.DMA,
            pltpu.SemaphoreType.DMA,
            pltpu.SemaphoreType.DMA,
        ],
    )(functools.partial(_edge_body, nh=nh))


def kernel(atom_features, edge_index, Ws, att_s, att_d, bs, Wo, att_so, att_do, bo):
    x_pad = jnp.pad(atom_features, ((0, NPAD - NNODE), (0, 0)))
    loop_idx = jnp.arange(NNODE, dtype=jnp.int32)
    npad_e = EPAD - NEDGE - NNODE
    pad_src = (jnp.arange(npad_e, dtype=jnp.int32) * 37) % NNODE
    pad_dst = NNODE + (jnp.arange(npad_e, dtype=jnp.int32) % (NPAD - NNODE))
    src = jnp.concatenate([edge_index[0], loop_idx, pad_src]).reshape(EROWS, CW)
    dst = jnp.concatenate([edge_index[1], loop_idx, pad_dst]).reshape(EROWS, CW)

    h_all, as_all, ad_all, c_all = _stage_a(x_pad, Ws, att_s.reshape(NHEAD, 1, F),
                                            att_d.reshape(NHEAD, 1, F))
    parts = _make_edge_pass(NHEAD)(
        h_all[0], h_all[1], h_all[2], h_all[3],
        as_all[:, 0, :], ad_all[:, 0, :], c_all.reshape(NHEAD, 2 * LANES),
        src, dst)[0]
    wo4 = Wo.reshape(NHEAD, F, F)
    h2, as2, ad2, c2 = _stage_b(parts, bs, wo4, att_so.reshape(1, F),
                                att_do.reshape(1, F))
    p2 = _make_edge_pass(1)(h2, as2, ad2, c2.reshape(1, 2 * LANES),
                            src, dst)[0]
    return _stage_c(p2.reshape(NC, NPAD, F), bo.reshape(1, F))


# speedup vs baseline: 2.0344x; 1.0087x over previous
import functools

import jax
import jax.numpy as jnp
import numpy as np
from jax import lax
from jax.experimental import pallas as pl
from jax.experimental.pallas import tpu as pltpu
from jax.experimental.pallas import tpu_sc as plsc

NNODE = 10000
NPAD = 10240
NEDGE = 320000
EPAD = 344064
CW = 64
EROWS = EPAD // CW
BLK = 8
F = 128
NHEAD = 4
NC, NS, LANES = 2, 16, 16
ROWS_PER_TILE = NPAD // NS
P1_ROWS = EROWS // NS
P2_ROWS = EROWS // (NC * NS)


def _stage_a_body(x_ref, w_ref, aw_s_ref, aw_d_ref, h_ref, as_ref, ad_ref, c_ref):
    hb = jnp.dot(x_ref[...], w_ref[0], preferred_element_type=jnp.float32)
    h_ref[0] = hb
    asb = jnp.sum(hb * aw_s_ref[0, 0], axis=1)
    adb = jnp.sum(hb * aw_d_ref[0, 0], axis=1)
    as_ref[0, 0] = asb
    ad_ref[0, 0] = adb

    @pl.when(pl.program_id(1) == 0)
    def _():
        c_ref[...] = jnp.full((1, 2, 16), -1e30, jnp.float32)

    c_ref[0, 0, :] = jnp.maximum(c_ref[0, 0, :], jnp.max(asb))
    c_ref[0, 1, :] = jnp.maximum(c_ref[0, 1, :], jnp.max(adb))


_stage_a = pl.pallas_call(
    _stage_a_body,
    grid=(NHEAD, NPAD // 1024),
    in_specs=[
        pl.BlockSpec((1024, F), lambda h, i: (i, 0)),
        pl.BlockSpec((1, F, F), lambda h, i: (h, 0, 0)),
        pl.BlockSpec((1, 1, F), lambda h, i: (h, 0, 0)),
        pl.BlockSpec((1, 1, F), lambda h, i: (h, 0, 0)),
    ],
    out_specs=[
        pl.BlockSpec((1, 1024, F), lambda h, i: (h, i, 0)),
        pl.BlockSpec((1, 1, 1024), lambda h, i: (h, 0, i)),
        pl.BlockSpec((1, 1, 1024), lambda h, i: (h, 0, i)),
        pl.BlockSpec((1, 2, 16), lambda h, i: (h, 0, 0)),
    ],
    out_shape=[
        jax.ShapeDtypeStruct((NHEAD, NPAD, F), jnp.float32),
        jax.ShapeDtypeStruct((NHEAD, 1, NPAD), jnp.float32),
        jax.ShapeDtypeStruct((NHEAD, 1, NPAD), jnp.float32),
        jax.ShapeDtypeStruct((NHEAD, 2, 16), jnp.float32),
    ],
)


def _stage_b_body(p_ref, bs_ref, wo_ref, so_ref, do_ref,
                  h2_ref, as2_ref, ad2_ref, c_ref):
    acc = jnp.zeros((1024, F), jnp.float32)
    for h in range(NHEAD):
        ph = p_ref[h, 0] + p_ref[h, 1] + bs_ref[h][None, :]
        acc = acc + jnp.dot(ph, wo_ref[h], preferred_element_type=jnp.float32)
    h2_ref[...] = acc
    as2b = jnp.sum(acc * so_ref[0], axis=1)
    ad2b = jnp.sum(acc * do_ref[0], axis=1)
    as2_ref[0] = as2b
    ad2_ref[0] = ad2b

    @pl.when(pl.program_id(0) == 0)
    def _():
        c_ref[...] = jnp.full((2, 16), -1e30, jnp.float32)

    c_ref[0, :] = jnp.maximum(c_ref[0, :], jnp.max(as2b))
    c_ref[1, :] = jnp.maximum(c_ref[1, :], jnp.max(ad2b))


_stage_b = pl.pallas_call(
    _stage_b_body,
    grid=(NPAD // 1024,),
    in_specs=[pl.BlockSpec((NHEAD, NC, 1024, F), lambda i: (0, 0, i, 0))] + [
        pl.BlockSpec((NHEAD, F), lambda i: (0, 0)),
        pl.BlockSpec((NHEAD, F, F), lambda i: (0, 0, 0)),
        pl.BlockSpec((1, F), lambda i: (0, 0)),
        pl.BlockSpec((1, F), lambda i: (0, 0)),
    ],
    out_specs=[
        pl.BlockSpec((1024, F), lambda i: (i, 0)),
        pl.BlockSpec((1, 1024), lambda i: (0, i)),
        pl.BlockSpec((1, 1024), lambda i: (0, i)),
        pl.BlockSpec((2, 16), lambda i: (0, 0)),
    ],
    out_shape=[
        jax.ShapeDtypeStruct((NPAD, F), jnp.float32),
        jax.ShapeDtypeStruct((1, NPAD), jnp.float32),
        jax.ShapeDtypeStruct((1, NPAD), jnp.float32),
        jax.ShapeDtypeStruct((2, 16), jnp.float32),
    ],
)


def _stage_c_body(p_ref, bo_ref, out_ref):
    out_ref[...] = p_ref[0] + p_ref[1] + bo_ref[0][None, :]


_stage_c = pl.pallas_call(
    _stage_c_body,
    grid=(NNODE // 1000,),
    in_specs=[
        pl.BlockSpec((NC, 1000, F), lambda i: (0, i, 0)),
        pl.BlockSpec((1, F), lambda i: (0, 0)),
    ],
    out_specs=pl.BlockSpec((1000, F), lambda i: (i, 0)),
    out_shape=jax.ShapeDtypeStruct((NNODE, F), jnp.float32),
)


def _edge_body(*refs, nh):
    h_refs = refs[:nh]
    (asv_hbm, adv_hbm, cvec_hbm, src_hbm, dst_hbm, out_hbm, p_hbm,
     nb1, nb2, sstage, dstage, pstage, sc_a, sc_b, wbuf, cbuf,
     out_sh, den_sh, sem_a, sem_b, sem_sca, sem_scb, sem_s, sem_pw,
     sem_st) = refs[nh:]
    cid = lax.axis_index("c")
    sid = lax.axis_index("s")
    for hd in range(nh):
        _edge_one_head(h_refs[hd], asv_hbm.at[hd], adv_hbm.at[hd],
                       cvec_hbm.at[hd], src_hbm, dst_hbm,
                       out_hbm.at[hd], p_hbm.at[hd],
                       nb1, nb2, sstage, dstage, pstage, sc_a,
                       sc_b, wbuf, cbuf, out_sh, den_sh, sem_a, sem_b,
                       sem_sca, sem_scb, sem_s, sem_pw, sem_st, cid, sid)


def _edge_one_head(h_hbm, asv_hbm, adv_hbm, cvec_hbm, src_hbm, dst_hbm,
                   out_hbm, p_hbm,
                   nb1, nb2, sstage, dstage, pstage, sc_a, sc_b,
                   wbuf, cbuf,
                   out_sh, den_sh, sem_a, sem_b, sem_sca, sem_scb, sem_s,
                   sem_pw, sem_st, cid, sid):
    nvec = CW // LANES

    pltpu.sync_copy(asv_hbm, nb1)
    pltpu.sync_copy(adv_hbm, nb2)
    pltpu.sync_copy(cvec_hbm, cbuf)
    cshift = cbuf[pl.ds(0, LANES)] + cbuf[pl.ds(LANES, LANES)]

    zv = jnp.zeros((LANES,), jnp.float32)

    def _zrow(k, _):
        for l in range(8):
            sc_a[k, pl.ds(l * LANES, LANES)] = zv
        return 0

    lax.fori_loop(0, CW, _zrow, 0)
    for l in range(CW // LANES):
        wbuf[pl.ds(l * LANES, LANES)] = zv
    for k in range(ROWS_PER_TILE // CW):
        off = sid * ROWS_PER_TILE + k * CW
        pltpu.sync_copy(sc_a, out_sh.at[pl.ds(off, CW)])
    for k in range(ROWS_PER_TILE // CW):
        off = sid * ROWS_PER_TILE + k * CW
        pltpu.sync_copy(wbuf, den_sh.at[pl.ds(off, CW)])
    plsc.subcore_barrier()

    p1_base = sid * P1_ROWS
    p1_n = P1_ROWS // BLK
    pltpu.async_copy(src_hbm.at[pl.ds(p1_base, BLK)],
                     sstage.at[pl.ds(0, BLK)], sem_st)
    pltpu.async_copy(dst_hbm.at[pl.ds(p1_base, BLK)],
                     dstage.at[pl.ds(0, BLK)], sem_st)

    def _p1(blk, _):
        row8 = pl.multiple_of(p1_base + blk * BLK, 8)
        par = (blk % 2) * BLK
        pltpu.make_async_copy(src_hbm.at[pl.ds(row8, BLK)],
                              sstage.at[pl.ds(par, BLK)], sem_st).wait()
        pltpu.make_async_copy(dst_hbm.at[pl.ds(row8, BLK)],
                              dstage.at[pl.ds(par, BLK)], sem_st).wait()

        @pl.when(blk + 1 < p1_n)
        def _prefetch():
            row8n = pl.multiple_of(p1_base + (blk + 1) * BLK, 8)
            parn = ((blk + 1) % 2) * BLK
            pltpu.async_copy(src_hbm.at[pl.ds(row8n, BLK)],
                             sstage.at[pl.ds(parn, BLK)], sem_st)
            pltpu.async_copy(dst_hbm.at[pl.ds(row8n, BLK)],
                             dstage.at[pl.ds(parn, BLK)], sem_st)

        cps = []
        for jj in range(BLK):
            for v in range(nvec):
                sv = sstage[par + jj, pl.ds(v * LANES, LANES)]
                dv = dstage[par + jj, pl.ds(v * LANES, LANES)]
                e = (plsc.load_gather(nb1, [sv]) +
                     plsc.load_gather(nb2, [dv]))
                e = jnp.where(e > 0, e, 0.2 * e)
                pstage[jj, pl.ds(v * LANES, LANES)] = jnp.exp(e - cshift)
            cps.append(pltpu.async_copy(pstage.at[jj],
                                        den_sh.at[dstage.at[par + jj]],
                                        sem_s, add=True))
        cps.append(pltpu.async_copy(pstage.at[pl.ds(0, BLK)],
                                    p_hbm.at[pl.ds(row8, BLK)], sem_pw))
        for cp in cps:
            cp.wait()
        return 0

    lax.fori_loop(0, p1_n, _p1, 0)
    plsc.subcore_barrier()
    pltpu.sync_copy(den_sh, nb1)

    gbufs = (sc_a, sc_b)
    rbufs = (sc_a, sc_b)
    sems = (sem_a, sem_b)
    scsems = (sem_sca, sem_scb)
    p2_base = cid * (EROWS // 2) + sid * P2_ROWS
    p2_n = P2_ROWS // BLK
    for hbm, st in ((src_hbm, sstage), (dst_hbm, dstage), (p_hbm, pstage)):
        pltpu.async_copy(hbm.at[pl.ds(p2_base, BLK)],
                         st.at[pl.ds(0, BLK)], sem_st)

    def _p2(blk, _):
        row8 = pl.multiple_of(p2_base + blk * BLK, 8)
        par = (blk % 2) * BLK
        for hbm, st in ((src_hbm, sstage), (dst_hbm, dstage), (p_hbm, pstage)):
            pltpu.make_async_copy(hbm.at[pl.ds(row8, BLK)],
                                  st.at[pl.ds(par, BLK)], sem_st).wait()

        @pl.when(blk + 1 < p2_n)
        def _prefetch():
            row8n = pl.multiple_of(p2_base + (blk + 1) * BLK, 8)
            parn = ((blk + 1) % 2) * BLK
            for hbm, st in ((src_hbm, sstage), (dst_hbm, dstage),
                            (p_hbm, pstage)):
                pltpu.async_copy(hbm.at[pl.ds(row8n, BLK)],
                                 st.at[pl.ds(parn, BLK)], sem_st)

        sc_pending = [None, None]
        cp = pltpu.async_copy(h_hbm.at[sstage.at[par]], gbufs[0], sems[0])
        for jj in range(BLK):
            b = jj % 2
            if jj + 1 < BLK:
                nb_ = (jj + 1) % 2
                if sc_pending[nb_] is not None:
                    sc_pending[nb_].wait()
                    sc_pending[nb_] = None
                cp_next = pltpu.async_copy(h_hbm.at[sstage.at[par + jj + 1]],
                                           gbufs[nb_], sems[nb_])
            buf = rbufs[b]
            for v in range(nvec):
                dv = dstage[par + jj, pl.ds(v * LANES, LANES)]
                dn = plsc.load_gather(nb1, [dv])
                pv = pstage[par + jj, pl.ds(v * LANES, LANES)]
                wbuf[pl.ds(v * LANES, LANES)] = pv / (dn + 1e-16)
            cp.wait()

            def _scale(k4, _c):
                for r in range(4):
                    k = k4 * 4 + r
                    wk = plsc.load_gather(
                        wbuf, [jnp.full((LANES,), r, jnp.int32) + k4 * 4])
                    for l in range(8):
                        buf[k, pl.ds(l * LANES, LANES)] = (
                            buf[k, pl.ds(l * LANES, LANES)] * wk)
                return 0

            lax.fori_loop(0, CW // 4, _scale, 0)
            sc_pending[b] = pltpu.async_copy(buf, out_sh.at[dstage.at[par + jj]],
                                             scsems[b], add=True)
            if jj + 1 < BLK:
                cp = cp_next
        for d in sc_pending:
            if d is not None:
                d.wait()
        return 0

    lax.fori_loop(0, P2_ROWS // BLK, _p2, 0)
    plsc.subcore_barrier()

    for k in range(ROWS_PER_TILE // 128):
        off = sid * ROWS_PER_TILE + k * 128
        pltpu.sync_copy(out_sh.at[pl.ds(off, 128)],
                        out_hbm.at[cid, pl.ds(off, 128)])


@functools.cache
def _make_edge_pass(nh):
    return functools.partial(
        pl.kernel,
        mesh=plsc.VectorSubcoreMesh(core_axis_name="c", subcore_axis_name="s"),
        out_type=[
            jax.ShapeDtypeStruct((nh, NC, NPAD, F), jnp.float32),
            jax.ShapeDtypeStruct((nh, EROWS, CW), jnp.float32),
        ],
        compiler_params=pltpu.CompilerParams(needs_layout_passes=False),
        scratch_types=[
            pltpu.VMEM((NPAD,), jnp.float32),
            pltpu.VMEM((NPAD,), jnp.float32),
            pltpu.VMEM((2 * BLK, CW), jnp.int32),
            pltpu.VMEM((2 * BLK, CW), jnp.int32),
            pltpu.VMEM((2 * BLK, CW), jnp.float32),
            pltpu.VMEM((CW, F), jnp.float32),
            pltpu.VMEM((CW, F), jnp.float32),
            pltpu.VMEM((CW,), jnp.float32),
            pltpu.VMEM((2 * LANES,), jnp.float32),
            pltpu.VMEM_SHARED((NPAD, F), jnp.float32),
            pltpu.VMEM_SHARED((NPAD,), jnp.float32),
            pltpu.SemaphoreType.DMA,
            pltpu.SemaphoreType.DMA,
            pltpu.SemaphoreType.DMA,
            pltpu.SemaphoreType.DMA,
            pltpu.SemaphoreType.DMA,
            pltpu.SemaphoreType.DMA,
            pltpu.SemaphoreType.DMA,
        ],
    )(functools.partial(_edge_body, nh=nh))


def kernel(atom_features, edge_index, Ws, att_s, att_d, bs, Wo, att_so, att_do, bo):
    x_pad = jnp.pad(atom_features, ((0, NPAD - NNODE), (0, 0)))
    loop_idx = jnp.arange(NNODE, dtype=jnp.int32)
    npad_e = EPAD - NEDGE - NNODE
    pad_src = (jnp.arange(npad_e, dtype=jnp.int32) * 37) % NNODE
    pad_dst = NNODE + (jnp.arange(npad_e, dtype=jnp.int32) % (NPAD - NNODE))
    src = jnp.concatenate([edge_index[0], loop_idx, pad_src]).reshape(EROWS, CW)
    dst = jnp.concatenate([edge_index[1], loop_idx, pad_dst]).reshape(EROWS, CW)

    h_all, as_all, ad_all, c_all = _stage_a(x_pad, Ws, att_s.reshape(NHEAD, 1, F),
                                            att_d.reshape(NHEAD, 1, F))
    parts = _make_edge_pass(NHEAD)(
        h_all[0], h_all[1], h_all[2], h_all[3],
        as_all[:, 0, :], ad_all[:, 0, :], c_all.reshape(NHEAD, 2 * LANES),
        src, dst)[0]
    wo4 = Wo.reshape(NHEAD, F, F)
    h2, as2, ad2, c2 = _stage_b(parts, bs, wo4, att_so.reshape(1, F),
                                att_do.reshape(1, F))
    p2 = _make_edge_pass(1)(h2, as2, ad2, c2.reshape(1, 2 * LANES),
                            src, dst)[0]
    return _stage_c(p2.reshape(NC, NPAD, F), bo.reshape(1, F))
